# baseline (speedup 1.0000x reference)
import dataclasses
import functools

import jax
import jax.numpy as jnp
from jax import lax
from jax.experimental import pallas as pl
from jax.experimental.pallas import tpu as pltpu
from jax.experimental.pallas import tpu_sc as plsc

N = 10000
E = 320000
D = 128

NC = 2
NS = 16
NW = NC * NS
NP = 10112
EPT = E // NW
CH = 125
NCHUNK = EPT // CH
IB = 8
NBLK = NCHUNK // IB
GP = EPT // 16

_mesh = plsc.VectorSubcoreMesh(core_axis_name="c", subcore_axis_name="s")

_cp_nolayout = pltpu.CompilerParams()
if "needs_layout_passes" in pltpu.CompilerParams.__dataclass_fields__:
    _cp_nolayout = dataclasses.replace(_cp_nolayout, needs_layout_passes=False)


@functools.partial(
    pl.kernel,
    compiler_params=_cp_nolayout,
    out_type=(
        jax.ShapeDtypeStruct((NW, N), jnp.float32),
        jax.ShapeDtypeStruct((NW, N), jnp.float32),
    ),
    mesh=_mesh,
    scratch_types=[
        pltpu.VMEM((N,), jnp.float32),
        pltpu.VMEM((N,), jnp.float32),
        pltpu.VMEM((EPT,), jnp.int32),
        pltpu.VMEM((EPT,), jnp.int32),
    ],
)
def _sc_degrees(src_hbm, dst_hbm, zeros_hbm, do_hbm, di_hbm,
                hso, hsi, src_v, dst_v):
    cid = lax.axis_index("c")
    sid = lax.axis_index("s")
    wid = sid * NC + cid

    pltpu.sync_copy(zeros_hbm, hso)
    pltpu.sync_copy(zeros_hbm, hsi)
    pltpu.sync_copy(src_hbm.at[pl.ds(wid * EPT, EPT)], src_v)
    pltpu.sync_copy(dst_hbm.at[pl.ds(wid * EPT, EPT)], dst_v)

    ones = jnp.full((16,), 1.0, dtype=jnp.float32)

    @pl.loop(0, GP)
    def _(k):
        iv = src_v[pl.ds(k * 16, 16)]
        plsc.addupdate_scatter(hso, [iv], ones)
        jv = dst_v[pl.ds(k * 16, 16)]
        plsc.addupdate_scatter(hsi, [jv], ones)

    pltpu.sync_copy(hso, do_hbm.at[wid])
    pltpu.sync_copy(hsi, di_hbm.at[wid])


@functools.partial(
    pl.kernel,
    out_type=jax.ShapeDtypeStruct((NC, NP, D), jnp.float32),
    mesh=_mesh,
    scratch_types=[
        pltpu.VMEM_SHARED((NP, D), jnp.float32),
        pltpu.VMEM((IB, 1, CH), jnp.int32),
        pltpu.VMEM((IB, 1, CH), jnp.int32),
        pltpu.VMEM((IB, 1, CH), jnp.int32),
        pltpu.VMEM((IB, 1, CH), jnp.int32),
        pltpu.VMEM((CH, D), jnp.float32),
        pltpu.VMEM((CH, D), jnp.float32),
        pltpu.SemaphoreType.DMA,
        pltpu.SemaphoreType.DMA,
        pltpu.SemaphoreType.DMA,
        pltpu.SemaphoreType.DMA,
    ],
)
def _sc_edge_agg(h_hbm, src_hbm, dst_hbm, zeros_hbm, out_hbm,
                 sagg, srcA, dstA, srcB, dstB, rows0, rows1,
                 semg0, semg1, semiA, semiB):
    cid = lax.axis_index("c")
    sid = lax.axis_index("s")
    wid = sid * NC + cid
    rows = NP // NS

    pltpu.sync_copy(zeros_hbm, rows0)
    base = sid * rows
    for off, ln in ((0, 120), (120, 120), (240, 120), (360, 120),
                    (480, 120), (600, 32)):
        pltpu.sync_copy(rows0.at[pl.ds(0, ln)],
                        sagg.at[pl.ds(base + off, ln)])
    plsc.subcore_barrier()

    rowb = (rows0, rows1)
    semb = (semg0, semg1)

    def idx_fetch(blk, sbuf, dbuf, sem):
        pltpu.async_copy(src_hbm.at[wid].at[blk], sbuf, sem)
        pltpu.async_copy(dst_hbm.at[wid].at[blk], dbuf, sem)

    def idx_wait(blk, sbuf, dbuf, sem):
        pltpu.make_async_copy(src_hbm.at[wid].at[blk], sbuf, sem).wait()
        pltpu.make_async_copy(dst_hbm.at[wid].at[blk], dbuf, sem).wait()

    def gather(sbuf, j, b):
        return pltpu.async_copy(h_hbm.at[sbuf.at[j].at[0]], rowb[b], semb[b])

    def scatter(dbuf, j, b):
        pltpu.sync_copy(rowb[b], sagg.at[dbuf.at[j].at[0]], add=True)

    def run_block(blk, sbuf, dbuf, sem):
        idx_wait(blk, sbuf, dbuf, sem)
        cp = gather(sbuf, 0, 0)
        for j in range(IB):
            cp.wait()
            if j + 1 < IB:
                cp = gather(sbuf, j + 1, (j + 1) % 2)
            scatter(dbuf, j, j % 2)

    idx_fetch(0, srcA, dstA, semiA)
    idx_fetch(1, srcB, dstB, semiB)

    @pl.loop(0, NBLK, step=2)
    def _(t):
        run_block(t, srcA, dstA, semiA)

        @pl.when(t + 2 < NBLK)
        def _():
            idx_fetch(t + 2, srcA, dstA, semiA)

        run_block(t + 1, srcB, dstB, semiB)

        @pl.when(t + 3 < NBLK)
        def _():
            idx_fetch(t + 3, srcB, dstB, semiB)

    plsc.subcore_barrier()
    pltpu.sync_copy(sagg.at[pl.ds(sid * rows, rows)],
                    out_hbm.at[cid].at[pl.ds(sid * rows, rows)])


BN = 1264
NBLK_TC = NP // BN


def _norm_from_dp(dp_blk):
    d = jnp.sum(dp_blk, axis=1, keepdims=True)
    return jnp.where(d > 0.0, lax.rsqrt(jnp.where(d > 0.0, d, 1.0)), 1.0)


def _dp_spec():
    return pl.BlockSpec((BN, NW), lambda i: (i, 0))


def _scale_mm_body(x_ref, do_ref, w_ref, o_ref):
    xs = x_ref[...] * _norm_from_dp(do_ref[...])
    o_ref[...] = jnp.dot(xs, w_ref[...])


def _tc_scale_mm(x, do_p, W):
    return pl.pallas_call(
        _scale_mm_body,
        grid=(NBLK_TC,),
        in_specs=[pl.BlockSpec((BN, D), lambda i: (i, 0)),
                  _dp_spec(),
                  pl.BlockSpec((D, D), lambda i: (0, 0))],
        out_specs=pl.BlockSpec((BN, D), lambda i: (i, 0)),
        out_shape=jax.ShapeDtypeStruct((NP, D), jnp.float32),
    )(x, do_p, W)


def _layer_body(agg_a, agg_b, do_ref, di_ref, b_ref, w_ref, o_ref):
    agg = agg_a[0] + agg_b[0]
    h = jnp.maximum(agg * _norm_from_dp(di_ref[...]) + b_ref[...], 0.0)
    hs = h * _norm_from_dp(do_ref[...])
    o_ref[...] = jnp.dot(hs, w_ref[...])


def _tc_layer_mid(agg_p, do_p, di_p, b1, W2):
    agg_spec_a = pl.BlockSpec((1, BN, D), lambda i: (0, i, 0))
    agg_spec_b = pl.BlockSpec((1, BN, D), lambda i: (1, i, 0))
    return pl.pallas_call(
        _layer_body,
        grid=(NBLK_TC,),
        in_specs=[agg_spec_a, agg_spec_b, _dp_spec(), _dp_spec(),
                  pl.BlockSpec((1, D), lambda i: (0, 0)),
                  pl.BlockSpec((D, D), lambda i: (0, 0))],
        out_specs=pl.BlockSpec((BN, D), lambda i: (i, 0)),
        out_shape=jax.ShapeDtypeStruct((NP, D), jnp.float32),
    )(agg_p, agg_p, do_p, di_p, b1.reshape(1, D), W2)


def _head_body(agg_a, agg_b, di_ref, b2_ref, w3_ref, b3_ref,
               w4_ref, b4_ref, w5_ref, b5_ref, o_ref, acc):
    i = pl.program_id(0)
    agg = agg_a[0] + agg_b[0]
    h2 = jnp.maximum(agg * _norm_from_dp(di_ref[...]) + b2_ref[...], 0.0)
    row = i * BN + lax.broadcasted_iota(jnp.int32, (BN, 1), 0)
    h2 = jnp.where(row < N, h2, 0.0)
    part = jnp.sum(h2, axis=0, keepdims=True)

    @pl.when(i == 0)
    def _():
        acc[0:1, :] = part

    @pl.when(i > 0)
    def _():
        acc[0:1, :] = acc[0:1, :] + part

    @pl.when(i == NBLK_TC - 1)
    def _():
        hg = acc[0:1, :] * (1.0 / N)
        hg = jnp.maximum(jnp.dot(hg, w3_ref[...]) + b3_ref[...], 0.0)
        hg = jnp.maximum(jnp.dot(hg, w4_ref[...]) + b4_ref[...], 0.0)
        o_ref[...] = jnp.dot(hg, w5_ref[...]) + b5_ref[...]


def _tc_head(agg_p, di_p, b2, W3, b3, W4, b4, W5, b5):
    agg_spec_a = pl.BlockSpec((1, BN, D), lambda i: (0, i, 0))
    agg_spec_b = pl.BlockSpec((1, BN, D), lambda i: (1, i, 0))
    full = lambda s: pl.BlockSpec(s, lambda i: tuple(0 for _ in s))
    return pl.pallas_call(
        _head_body,
        grid=(NBLK_TC,),
        in_specs=[agg_spec_a, agg_spec_b, _dp_spec(),
                  full((1, D)), full((D, 2 * D)), full((1, 2 * D)),
                  full((2 * D, D)), full((1, D)), full((D, 1)),
                  full((1, 1))],
        out_specs=pl.BlockSpec((1, 1), lambda i: (0, 0)),
        out_shape=jax.ShapeDtypeStruct((1, 1), jnp.float32),
        scratch_shapes=[pltpu.VMEM((8, D), jnp.float32)],
    )(agg_p, agg_p, di_p, b2.reshape(1, D), W3, b3.reshape(1, -1),
      W4, b4.reshape(1, -1), W5, b5.reshape(1, 1))


@jax.jit
def _run(x, edge_index, W1, b1, W2, b2, W3, b3, W4, b4, W5, b5):
    src = edge_index[0].astype(jnp.int32)
    dst = edge_index[1].astype(jnp.int32)
    src5 = src.reshape(NW, NBLK, IB, 1, CH)
    dst5 = dst.reshape(NW, NBLK, IB, 1, CH)
    zeros_ch = jnp.zeros((CH, D), jnp.float32)
    zerosN = jnp.zeros((N,), jnp.float32)

    do_p, di_p = _sc_degrees(src, dst, zerosN)
    do_t = do_p.T
    di_t = di_p.T
    h1p = _tc_scale_mm(x, do_t, W1)
    agg1 = _sc_edge_agg(h1p, src5, dst5, zeros_ch)
    h2p = _tc_layer_mid(agg1, do_t, di_t, b1, W2)
    agg2 = _sc_edge_agg(h2p, src5, dst5, zeros_ch)
    return _tc_head(agg2, di_t, b2, W3, b3, W4, b4, W5, b5)


def kernel(x, edge_index, W1, b1, W2, b2, W3, b3, W4, b4, W5, b5):
    return _run(x, edge_index, W1, b1, W2, b2, W3, b3, W4, b4, W5, b5)


# speedup vs baseline: 1.0141x; 1.0141x over previous
import dataclasses
import functools

import jax
import jax.numpy as jnp
from jax import lax
from jax.experimental import pallas as pl
from jax.experimental.pallas import tpu as pltpu
from jax.experimental.pallas import tpu_sc as plsc

N = 10000
E = 320000
D = 128

NC = 2
NS = 16
NW = NC * NS
NP = 10112
EPT = E // NW
CH = 125
NCHUNK = EPT // CH
IB = 8
NBLK = NCHUNK // IB
GP = EPT // 16

_mesh = plsc.VectorSubcoreMesh(core_axis_name="c", subcore_axis_name="s")

_cp_nolayout = pltpu.CompilerParams()
if "needs_layout_passes" in pltpu.CompilerParams.__dataclass_fields__:
    _cp_nolayout = dataclasses.replace(_cp_nolayout, needs_layout_passes=False)


@functools.partial(
    pl.kernel,
    compiler_params=_cp_nolayout,
    out_type=(
        jax.ShapeDtypeStruct((NW, N), jnp.float32),
        jax.ShapeDtypeStruct((NW, N), jnp.float32),
    ),
    mesh=_mesh,
    scratch_types=[
        pltpu.VMEM((N,), jnp.float32),
        pltpu.VMEM((N,), jnp.float32),
        pltpu.VMEM((EPT,), jnp.int32),
        pltpu.VMEM((EPT,), jnp.int32),
    ],
)
def _sc_degrees(edges_hbm, zeros_hbm, do_hbm, di_hbm,
                hso, hsi, src_v, dst_v):
    cid = lax.axis_index("c")
    sid = lax.axis_index("s")
    wid = sid * NC + cid

    pltpu.sync_copy(zeros_hbm, hso)
    pltpu.sync_copy(zeros_hbm, hsi)
    pltpu.sync_copy(edges_hbm.at[pl.ds(wid * EPT, EPT)], src_v)
    pltpu.sync_copy(edges_hbm.at[pl.ds(E + wid * EPT, EPT)], dst_v)

    ones = jnp.full((16,), 1.0, dtype=jnp.float32)

    @pl.loop(0, GP)
    def _(k):
        iv = src_v[pl.ds(k * 16, 16)]
        plsc.addupdate_scatter(hso, [iv], ones)
        jv = dst_v[pl.ds(k * 16, 16)]
        plsc.addupdate_scatter(hsi, [jv], ones)

    pltpu.sync_copy(hso, do_hbm.at[wid])
    pltpu.sync_copy(hsi, di_hbm.at[wid])


@functools.partial(
    pl.kernel,
    out_type=jax.ShapeDtypeStruct((NC, NP, D), jnp.float32),
    mesh=_mesh,
    scratch_types=[
        pltpu.VMEM_SHARED((NP, D), jnp.float32),
        pltpu.VMEM((IB, 1, CH), jnp.int32),
        pltpu.VMEM((IB, 1, CH), jnp.int32),
        pltpu.VMEM((IB, 1, CH), jnp.int32),
        pltpu.VMEM((IB, 1, CH), jnp.int32),
        pltpu.VMEM((CH, D), jnp.float32),
        pltpu.VMEM((CH, D), jnp.float32),
        pltpu.SemaphoreType.DMA,
        pltpu.SemaphoreType.DMA,
        pltpu.SemaphoreType.DMA,
        pltpu.SemaphoreType.DMA,
    ],
)
def _sc_edge_agg(h_hbm, src_hbm, dst_hbm, zeros_hbm, out_hbm,
                 sagg, srcA, dstA, srcB, dstB, rows0, rows1,
                 semg0, semg1, semiA, semiB):
    cid = lax.axis_index("c")
    sid = lax.axis_index("s")
    wid = sid * NC + cid
    rows = NP // NS

    pltpu.sync_copy(zeros_hbm, rows0)
    base = sid * rows
    for off, ln in ((0, 120), (120, 120), (240, 120), (360, 120),
                    (480, 120), (600, 32)):
        pltpu.sync_copy(rows0.at[pl.ds(0, ln)],
                        sagg.at[pl.ds(base + off, ln)])
    plsc.subcore_barrier()

    rowb = (rows0, rows1)
    semb = (semg0, semg1)

    def idx_fetch(blk, sbuf, dbuf, sem):
        pltpu.async_copy(src_hbm.at[wid].at[blk], sbuf, sem)
        pltpu.async_copy(dst_hbm.at[wid].at[blk], dbuf, sem)

    def idx_wait(blk, sbuf, dbuf, sem):
        pltpu.make_async_copy(src_hbm.at[wid].at[blk], sbuf, sem).wait()
        pltpu.make_async_copy(dst_hbm.at[wid].at[blk], dbuf, sem).wait()

    def gather(sbuf, j, b):
        return pltpu.async_copy(h_hbm.at[sbuf.at[j].at[0]], rowb[b], semb[b])

    def scatter(dbuf, j, b):
        pltpu.sync_copy(rowb[b], sagg.at[dbuf.at[j].at[0]], add=True)

    def run_block(blk, sbuf, dbuf, sem):
        idx_wait(blk, sbuf, dbuf, sem)
        cp = gather(sbuf, 0, 0)
        for j in range(IB):
            cp.wait()
            if j + 1 < IB:
                cp = gather(sbuf, j + 1, (j + 1) % 2)
            scatter(dbuf, j, j % 2)

    idx_fetch(0, srcA, dstA, semiA)
    idx_fetch(1, srcB, dstB, semiB)

    @pl.loop(0, NBLK, step=2)
    def _(t):
        run_block(t, srcA, dstA, semiA)

        @pl.when(t + 2 < NBLK)
        def _():
            idx_fetch(t + 2, srcA, dstA, semiA)

        run_block(t + 1, srcB, dstB, semiB)

        @pl.when(t + 3 < NBLK)
        def _():
            idx_fetch(t + 3, srcB, dstB, semiB)

    plsc.subcore_barrier()
    pltpu.sync_copy(sagg.at[pl.ds(sid * rows, rows)],
                    out_hbm.at[cid].at[pl.ds(sid * rows, rows)])


BN = 1264
NBLK_TC = NP // BN


def _norm_from_dp(dp_blk):
    d = jnp.sum(dp_blk, axis=1, keepdims=True)
    return jnp.where(d > 0.0, lax.rsqrt(jnp.where(d > 0.0, d, 1.0)), 1.0)


def _dp_spec():
    return pl.BlockSpec((BN, NW), lambda i: (i, 0))


def _scale_mm_body(x_ref, do_ref, w_ref, o_ref):
    xs = x_ref[...] * _norm_from_dp(do_ref[...])
    o_ref[...] = jnp.dot(xs, w_ref[...])


def _tc_scale_mm(x, do_p, W):
    return pl.pallas_call(
        _scale_mm_body,
        grid=(NBLK_TC,),
        in_specs=[pl.BlockSpec((BN, D), lambda i: (i, 0)),
                  _dp_spec(),
                  pl.BlockSpec((D, D), lambda i: (0, 0))],
        out_specs=pl.BlockSpec((BN, D), lambda i: (i, 0)),
        out_shape=jax.ShapeDtypeStruct((NP, D), jnp.float32),
    )(x, do_p, W)


def _layer_body(agg_a, agg_b, do_ref, di_ref, b_ref, w_ref, o_ref):
    agg = agg_a[0] + agg_b[0]
    h = jnp.maximum(agg * _norm_from_dp(di_ref[...]) + b_ref[...], 0.0)
    hs = h * _norm_from_dp(do_ref[...])
    o_ref[...] = jnp.dot(hs, w_ref[...])


def _tc_layer_mid(agg_p, do_p, di_p, b1, W2):
    agg_spec_a = pl.BlockSpec((1, BN, D), lambda i: (0, i, 0))
    agg_spec_b = pl.BlockSpec((1, BN, D), lambda i: (1, i, 0))
    return pl.pallas_call(
        _layer_body,
        grid=(NBLK_TC,),
        in_specs=[agg_spec_a, agg_spec_b, _dp_spec(), _dp_spec(),
                  pl.BlockSpec((1, D), lambda i: (0, 0)),
                  pl.BlockSpec((D, D), lambda i: (0, 0))],
        out_specs=pl.BlockSpec((BN, D), lambda i: (i, 0)),
        out_shape=jax.ShapeDtypeStruct((NP, D), jnp.float32),
    )(agg_p, agg_p, do_p, di_p, b1.reshape(1, D), W2)


def _head_body(agg_a, agg_b, di_ref, b2_ref, w3_ref, b3_ref,
               w4_ref, b4_ref, w5_ref, b5_ref, o_ref, acc):
    i = pl.program_id(0)
    agg = agg_a[0] + agg_b[0]
    h2 = jnp.maximum(agg * _norm_from_dp(di_ref[...]) + b2_ref[...], 0.0)
    row = i * BN + lax.broadcasted_iota(jnp.int32, (BN, 1), 0)
    h2 = jnp.where(row < N, h2, 0.0)
    part = jnp.sum(h2, axis=0, keepdims=True)

    @pl.when(i == 0)
    def _():
        acc[0:1, :] = part

    @pl.when(i > 0)
    def _():
        acc[0:1, :] = acc[0:1, :] + part

    @pl.when(i == NBLK_TC - 1)
    def _():
        hg = acc[0:1, :] * (1.0 / N)
        hg = jnp.maximum(jnp.dot(hg, w3_ref[...]) + b3_ref[...], 0.0)
        hg = jnp.maximum(jnp.dot(hg, w4_ref[...]) + b4_ref[...], 0.0)
        o_ref[...] = jnp.dot(hg, w5_ref[...]) + b5_ref[...]


def _tc_head(agg_p, di_p, b2, W3, b3, W4, b4, W5, b5):
    agg_spec_a = pl.BlockSpec((1, BN, D), lambda i: (0, i, 0))
    agg_spec_b = pl.BlockSpec((1, BN, D), lambda i: (1, i, 0))
    full = lambda s: pl.BlockSpec(s, lambda i: tuple(0 for _ in s))
    return pl.pallas_call(
        _head_body,
        grid=(NBLK_TC,),
        in_specs=[agg_spec_a, agg_spec_b, _dp_spec(),
                  full((1, D)), full((D, 2 * D)), full((1, 2 * D)),
                  full((2 * D, D)), full((1, D)), full((D, 1)),
                  full((1, 1))],
        out_specs=pl.BlockSpec((1, 1), lambda i: (0, 0)),
        out_shape=jax.ShapeDtypeStruct((1, 1), jnp.float32),
        scratch_shapes=[pltpu.VMEM((8, D), jnp.float32)],
    )(agg_p, agg_p, di_p, b2.reshape(1, D), W3, b3.reshape(1, -1),
      W4, b4.reshape(1, -1), W5, b5.reshape(1, 1))


@jax.jit
def _run(x, edge_index, W1, b1, W2, b2, W3, b3, W4, b4, W5, b5):
    ei = edge_index.astype(jnp.int32)
    edges_flat = ei.reshape(2 * E)
    src5 = ei[0].reshape(NW, NBLK, IB, 1, CH)
    dst5 = ei[1].reshape(NW, NBLK, IB, 1, CH)
    zeros_ch = jnp.zeros((CH, D), jnp.float32)
    zerosN = jnp.zeros((N,), jnp.float32)

    do_p, di_p = _sc_degrees(edges_flat, zerosN)
    do_t = do_p.T
    di_t = di_p.T
    h1p = _tc_scale_mm(x, do_t, W1)
    agg1 = _sc_edge_agg(h1p, src5, dst5, zeros_ch)
    h2p = _tc_layer_mid(agg1, do_t, di_t, b1, W2)
    agg2 = _sc_edge_agg(h2p, src5, dst5, zeros_ch)
    return _tc_head(agg2, di_t, b2, W3, b3, W4, b4, W5, b5)


def kernel(x, edge_index, W1, b1, W2, b2, W3, b3, W4, b4, W5, b5):
    return _run(x, edge_index, W1, b1, W2, b2, W3, b3, W4, b4, W5, b5)
